# baseline (speedup 1.0000x reference)
import jax
import jax.numpy as jnp
from jax import lax
from jax.experimental import pallas as pl
from jax.experimental.pallas import tpu as pltpu
from jax.experimental.pallas import tpu_sc as plsc

_N = 5000
_C = 81
_M = 28
_P = _M * _M
_NC = 2
_NS = 16
_NW = _NC * _NS
_CW = 128

_SCW = 20
_SCCOLS = _SCW * _CW
_TCCOLS = _N - _SCCOLS
_TCB = (_TCCOLS + _CW - 1) // _CW

_PG = 7
_PPG = _P // _PG
_NU = _SCW * _PG
_UPW = (_NU + _NW - 1) // _NW
_NBUF = 6


def _sigmoid(v):
    return 1.0 / (1.0 + jnp.exp(-v))


def _sc_body(x_hbm, lab_hbm, out_hbm, lab_v, *rest):
    abufs = rest[:_NBUF]
    obuf = rest[_NBUF]
    gsem, osem = rest[_NBUF + 1], rest[_NBUF + 2]
    wid = lax.axis_index("s") * _NC + lax.axis_index("c")
    iot = lax.iota(jnp.int32, 16)

    def unit_body(u, carry):
        unit = wid + u * _NW

        @pl.when(unit < _NU)
        def _():
            cc = unit % _SCW
            pg = unit // _SCW
            col0 = pl.multiple_of(cc * _CW, _CW)
            p0 = pl.multiple_of(pg * _PPG, _PPG)
            pltpu.sync_copy(lab_hbm.at[pl.ds(col0, _CW)], lab_v)

            def fetch(pp, buf):
                pltpu.async_copy(x_hbm.at[p0 + pp, :, pl.ds(col0, _CW)],
                                 buf, gsem)

            def wait_fetch(buf):
                pltpu.make_async_copy(x_hbm.at[0, :, pl.ds(0, _CW)],
                                      buf, gsem).wait()

            for pr in range(_NBUF - 1):
                fetch(pr, abufs[pr])

            def plane_body(pp, c2):
                def run(h):
                    buf = abufs[h]
                    wait_fetch(buf)

                    @pl.when(pp + _NBUF - 1 < _PPG)
                    def _():
                        fetch(pp + _NBUF - 1, abufs[(h + _NBUF - 1) % _NBUF])

                    for g in range(_CW // 16):
                        rows = lab_v[pl.ds(g * 16, 16)]
                        cols = g * 16 + iot
                        v = plsc.load_gather(buf, [rows, cols])
                        obuf[pp, pl.ds(g * 16, 16)] = _sigmoid(v)

                for h in range(_NBUF):
                    @pl.when(pp % _NBUF == h)
                    def _(h=h):
                        run(h)

                return c2

            lax.fori_loop(0, _PPG, plane_body, 0)
            pltpu.async_copy(obuf,
                             out_hbm.at[pl.ds(p0, _PPG), pl.ds(col0, _CW)],
                             osem)
            pltpu.make_async_copy(obuf,
                                  out_hbm.at[pl.ds(0, _PPG), pl.ds(0, _CW)],
                                  osem).wait()

        return carry

    lax.fori_loop(0, _UPW, unit_body, 0)


def _tc_body(lab_ref, x_ref, o_ref):
    lab = lab_ref[0]
    iota_c = lax.broadcasted_iota(jnp.int32, (_C, _CW), 0)
    mask = (lab[None, :] == iota_c).astype(jnp.float32)
    for b in range(_M):
        sel = jnp.sum(x_ref[0, b] * mask, axis=0)
        o_ref[0, b, :] = _sigmoid(sel)


def kernel(x, labels):
    lab32 = labels.astype(jnp.int32)
    xt4 = jnp.transpose(x, (2, 3, 1, 0))
    xt3 = xt4.reshape(_P, _C, _N)
    mesh = plsc.VectorSubcoreMesh(core_axis_name="c", subcore_axis_name="s")
    sck = pl.kernel(
        _sc_body,
        out_type=jax.ShapeDtypeStruct((_P, _SCCOLS), jnp.float32),
        mesh=mesh,
        compiler_params=pltpu.CompilerParams(needs_layout_passes=False),
        scratch_types=[pltpu.VMEM((_CW,), jnp.int32)]
        + [pltpu.VMEM((_C, _CW), jnp.float32) for _ in range(_NBUF)]
        + [pltpu.VMEM((_PPG, _CW), jnp.float32),
           pltpu.SemaphoreType.DMA, pltpu.SemaphoreType.DMA],
    )
    out_sc = sck(xt3, lab32)

    lab2d = lab32[None, :]
    out_tc = pl.pallas_call(
        _tc_body,
        grid=(_M, _TCB),
        in_specs=[
            pl.BlockSpec((1, _CW), lambda a, j: (0, _SCW + j)),
            pl.BlockSpec((1, _M, _C, _CW), lambda a, j: (a, 0, 0, _SCW + j)),
        ],
        out_specs=pl.BlockSpec((1, _M, _CW), lambda a, j: (a, 0, j)),
        out_shape=jax.ShapeDtypeStruct((_M, _M, _TCB * _CW), jnp.float32),
    )(lab2d, xt4)

    main_sc = jnp.transpose(out_sc.reshape(_M, _M, _SCCOLS), (2, 0, 1))
    main_tc = jnp.transpose(out_tc[:, :, :_TCCOLS], (2, 0, 1))
    out = jnp.concatenate([main_sc, main_tc], axis=0)
    return out[:, None]


# speedup vs baseline: 1.2077x; 1.2077x over previous
import jax
import jax.numpy as jnp
from jax import lax
from jax.experimental import pallas as pl
from jax.experimental.pallas import tpu as pltpu
from jax.experimental.pallas import tpu_sc as plsc

_N = 5000
_C = 81
_M = 28
_P = _M * _M
_NC = 2
_NS = 16
_NW = _NC * _NS
_CW = 128

_SCW = 16
_SCCOLS = _SCW * _CW
_TCCOLS = _N - _SCCOLS
_CWT = 512
_TCB = (_TCCOLS + _CWT - 1) // _CWT

_PG = 7
_PPG = _P // _PG
_NU = _SCW * _PG
_UPW = (_NU + _NW - 1) // _NW
_NBUF = 6


def _sigmoid(v):
    return 1.0 / (1.0 + jnp.exp(-v))


def _sc_body(x_hbm, lab_hbm, out_hbm, lab_v, *rest):
    abufs = rest[:_NBUF]
    obuf = rest[_NBUF]
    gsem, osem = rest[_NBUF + 1], rest[_NBUF + 2]
    wid = lax.axis_index("s") * _NC + lax.axis_index("c")
    iot = lax.iota(jnp.int32, 16)

    def unit_body(u, carry):
        unit = wid + u * _NW

        @pl.when(unit < _NU)
        def _():
            cc = unit % _SCW
            pg = unit // _SCW
            col0 = pl.multiple_of(cc * _CW, _CW)
            p0 = pl.multiple_of(pg * _PPG, _PPG)
            pltpu.sync_copy(lab_hbm.at[pl.ds(col0, _CW)], lab_v)

            def fetch(pp, buf):
                pltpu.async_copy(x_hbm.at[p0 + pp, :, pl.ds(col0, _CW)],
                                 buf, gsem)

            def wait_fetch(buf):
                pltpu.make_async_copy(x_hbm.at[0, :, pl.ds(0, _CW)],
                                      buf, gsem).wait()

            for pr in range(_NBUF - 1):
                fetch(pr, abufs[pr])

            def plane_body(pp, c2):
                def run(h):
                    buf = abufs[h]
                    wait_fetch(buf)

                    @pl.when(pp + _NBUF - 1 < _PPG)
                    def _():
                        fetch(pp + _NBUF - 1, abufs[(h + _NBUF - 1) % _NBUF])

                    for g in range(_CW // 16):
                        rows = lab_v[pl.ds(g * 16, 16)]
                        cols = g * 16 + iot
                        v = plsc.load_gather(buf, [rows, cols])
                        obuf[pp, pl.ds(g * 16, 16)] = _sigmoid(v)

                for h in range(_NBUF):
                    @pl.when(pp % _NBUF == h)
                    def _(h=h):
                        run(h)

                return c2

            lax.fori_loop(0, _PPG, plane_body, 0)
            pltpu.async_copy(obuf,
                             out_hbm.at[pl.ds(p0, _PPG), pl.ds(col0, _CW)],
                             osem)
            pltpu.make_async_copy(obuf,
                                  out_hbm.at[pl.ds(0, _PPG), pl.ds(0, _CW)],
                                  osem).wait()

        return carry

    lax.fori_loop(0, _UPW, unit_body, 0)


def _tc_body(lab_ref, x_ref, o_ref):
    lab = lab_ref[0]
    iota_c = lax.broadcasted_iota(jnp.int32, (_C, _CWT), 0)
    mask = (lab[None, :] == iota_c).astype(jnp.float32)
    sel = jnp.sum(x_ref[0] * mask[None], axis=1)
    o_ref[0] = _sigmoid(sel)


def kernel(x, labels):
    lab32 = labels.astype(jnp.int32)
    xt4 = jnp.transpose(x, (2, 3, 1, 0))
    xt3 = xt4.reshape(_P, _C, _N)
    mesh = plsc.VectorSubcoreMesh(core_axis_name="c", subcore_axis_name="s")
    sck = pl.kernel(
        _sc_body,
        out_type=jax.ShapeDtypeStruct((_P, _SCCOLS), jnp.float32),
        mesh=mesh,
        compiler_params=pltpu.CompilerParams(needs_layout_passes=False),
        scratch_types=[pltpu.VMEM((_CW,), jnp.int32)]
        + [pltpu.VMEM((_C, _CW), jnp.float32) for _ in range(_NBUF)]
        + [pltpu.VMEM((_PPG, _CW), jnp.float32),
           pltpu.SemaphoreType.DMA, pltpu.SemaphoreType.DMA],
    )
    out_sc = sck(xt3, lab32)

    lab2d = lab32[None, :]
    out_tc = pl.pallas_call(
        _tc_body,
        grid=(_M, _TCB),
        in_specs=[
            pl.BlockSpec((1, _CWT), lambda a, j: (0, _SCCOLS // _CWT + j)),
            pl.BlockSpec((1, _M, _C, _CWT),
                         lambda a, j: (a, 0, 0, _SCCOLS // _CWT + j)),
        ],
        out_specs=pl.BlockSpec((1, _M, _CWT), lambda a, j: (a, 0, j)),
        out_shape=jax.ShapeDtypeStruct((_M, _M, _TCB * _CWT), jnp.float32),
    )(lab2d, xt4)

    main_sc = jnp.transpose(out_sc.reshape(_M, _M, _SCCOLS), (2, 0, 1))
    main_tc = jnp.transpose(out_tc[:, :, :_TCCOLS], (2, 0, 1))
    out = jnp.concatenate([main_sc, main_tc], axis=0)
    return out[:, None]
